# baseline (speedup 1.0000x reference)
import functools

import jax
import jax.numpy as jnp
from jax import lax
from jax.experimental import pallas as pl
from jax.experimental.pallas import tpu as pltpu
from jax.experimental.pallas import tpu_sc as plsc

_N = 10000
_E = 320000
_DIN = 128
_HID = 256
_OUT2 = 64
_G = 64

_NP = 10240
_NW = 32
_K = 128
_EP = 327680
_NB = _EP // (_NW * _K)
_RPS = _NP // 16

_HIGH = lax.Precision.HIGHEST


def _sc_mesh():
    return plsc.VectorSubcoreMesh(core_axis_name="c", subcore_axis_name="s",
                                  num_cores=2, num_subcores=16)


_SC_PARAMS = pltpu.CompilerParams(use_tc_tiling_on_sc=False)


def _sc_aggregate(table, src3, dst3, zeros, d):
    hb = _NB // 2

    @functools.partial(
        pl.kernel,
        out_type=jax.ShapeDtypeStruct((2, _NP, d), jnp.float32),
        mesh=_sc_mesh(),
        scratch_types=[
            pltpu.VMEM((hb, _K), jnp.int32),
            pltpu.VMEM((hb, _K), jnp.int32),
            pltpu.VMEM((_K, d), jnp.float32),
            pltpu.VMEM((_K, d), jnp.float32),
            pltpu.VMEM_SHARED((_NP, d), jnp.float32),
            pltpu.SemaphoreType.DMA,
            pltpu.SemaphoreType.DMA,
            pltpu.SemaphoreType.DMA,
            pltpu.SemaphoreType.DMA,
        ],
        compiler_params=_SC_PARAMS,
    )
    def agg_kernel(table_h, src_h, dst_h, zeros_h, out_h,
                   sidx, didx, rows0, rows1, acc, sg0, sg1, ss0, ss1):
        c = lax.axis_index("c")
        s = lax.axis_index("s")
        wid = s * 2 + c
        pltpu.sync_copy(zeros_h.at[pl.ds(s * _RPS, _RPS)],
                        acc.at[pl.ds(s * _RPS, _RPS)])
        plsc.subcore_barrier()

        def gs(b, buf, sem):
            pltpu.make_async_copy(table_h.at[sidx.at[b]], buf, sem).start()

        def gw(b, buf, sem):
            pltpu.make_async_copy(table_h.at[sidx.at[b]], buf, sem).wait()

        def ss(b, buf, sem):
            pltpu.async_copy(buf, acc.at[didx.at[b]], sem, add=True)

        def sw(b, buf, sem):
            pltpu.make_async_copy(buf, acc.at[didx.at[b]], sem).wait()

        @pl.loop(0, 2)
        def _(h):
            pltpu.sync_copy(src_h.at[wid, pl.ds(h * hb, hb)], sidx)
            pltpu.sync_copy(dst_h.at[wid, pl.ds(h * hb, hb)], didx)
            gs(0, rows0, sg0)

            @pl.loop(0, hb, step=2)
            def _(b):
                gs(b + 1, rows1, sg1)
                gw(b, rows0, sg0)
                ss(b, rows0, ss0)
                gw(b + 1, rows1, sg1)
                ss(b + 1, rows1, ss1)
                sw(b, rows0, ss0)

                @pl.when(b + 2 < hb)
                def _():
                    gs(b + 2, rows0, sg0)
                sw(b + 1, rows1, ss1)

        plsc.subcore_barrier()
        pltpu.sync_copy(acc.at[pl.ds(s * _RPS, _RPS)],
                        out_h.at[c, pl.ds(s * _RPS, _RPS)])

    return agg_kernel(table, src3, dst3, zeros)


def _tc_prep(degp, x_pad):
    def body(degp_ref, x_ref, xs1_ref, dis_ref):
        deg = degp_ref[0, :, 0:1] + degp_ref[1, :, 0:1] + 1.0
        dis = lax.rsqrt(deg)
        row = lax.broadcasted_iota(jnp.int32, (_NP, 1), 0)
        dis = jnp.where(row < _N, dis, 0.0)
        dis_ref[...] = dis
        xs1_ref[...] = x_ref[...] * dis

    return pl.pallas_call(
        body,
        out_shape=(jax.ShapeDtypeStruct((_NP, _DIN), jnp.float32),
                   jax.ShapeDtypeStruct((_NP, 1), jnp.float32)),
    )(degp, x_pad)


def _tc_mid(acc1, xs1, dis, W1, b1, W2):
    def body(acc_ref, xs1_ref, dis_ref, w1_ref, b1_ref, w2_ref, xs2_ref):
        dis = dis_ref[...]
        ax = dis * (acc_ref[0] + acc_ref[1] + xs1_ref[...])
        h1 = jnp.dot(ax, w1_ref[...], precision=_HIGH,
                     preferred_element_type=jnp.float32) + b1_ref[...]
        h1 = jnp.maximum(h1, 0.0)
        m2 = jnp.dot(h1, w2_ref[...], precision=_HIGH,
                     preferred_element_type=jnp.float32)
        xs2_ref[...] = dis * m2

    return pl.pallas_call(
        body,
        out_shape=jax.ShapeDtypeStruct((_NP, _OUT2), jnp.float32),
    )(acc1, xs1, dis, W1, b1, W2)


def _tc_final(acc2, xs2, dis, b2, batch_row, fW1, fb1, fW2, fb2):
    def body(acc_ref, xs2_ref, dis_ref, b2_ref, batch_ref,
             fw1_ref, fb1_ref, fw2_ref, fb2_ref, pooled_ref, x4_ref):
        dis = dis_ref[...]
        h2 = dis * (acc_ref[0] + acc_ref[1] + xs2_ref[...]) + b2_ref[...]
        h2 = jnp.maximum(h2, 0.0)
        gid = lax.broadcasted_iota(jnp.int32, (_G, _NP), 0)
        oneh = (gid == batch_ref[...]).astype(jnp.float32)
        psum = jnp.dot(oneh, h2, precision=_HIGH,
                       preferred_element_type=jnp.float32)
        cnt = jnp.sum(oneh, axis=1, keepdims=True)
        pooled = psum / jnp.maximum(cnt, 1.0)
        pooled_ref[...] = pooled
        x1 = jnp.dot(pooled, fw1_ref[...], precision=_HIGH,
                     preferred_element_type=jnp.float32) + fb1_ref[...]
        x2 = jnp.maximum(x1, 0.0)
        x3 = jnp.dot(x2, fw2_ref[...], precision=_HIGH,
                     preferred_element_type=jnp.float32) + fb2_ref[...]
        x4_ref[...] = jnp.maximum(x3, 0.0)

    return pl.pallas_call(
        body,
        out_shape=(jax.ShapeDtypeStruct((_G, _OUT2), jnp.float32),
                   jax.ShapeDtypeStruct((_G, _HID), jnp.float32)),
    )(acc2, xs2, dis, b2, batch_row, fW1, fb1, fW2, fb2)


def kernel(x, edge_index, batch, W1, b1, W2, b2, fW1, fb1, fW2, fb2):
    src = edge_index[0].astype(jnp.int32)
    dst = edge_index[1].astype(jnp.int32)
    epad = _EP - _E
    padv = jnp.full((epad,), _N, jnp.int32)
    src3 = jnp.concatenate([src, padv]).reshape(_NW, _NB, _K)
    dst3 = jnp.concatenate([dst, padv]).reshape(_NW, _NB, _K)

    x_pad = jnp.pad(x, ((0, _NP - _N), (0, 0)))
    batch_row = jnp.concatenate(
        [batch.astype(jnp.int32), jnp.full((_NP - _N,), _G, jnp.int32)]
    ).reshape(1, _NP)

    zeros16 = jnp.zeros((_NP, 16), jnp.float32)
    zeros128 = jnp.zeros((_NP, _DIN), jnp.float32)
    zeros64 = jnp.zeros((_NP, _OUT2), jnp.float32)
    row16 = lax.broadcasted_iota(jnp.int32, (_NP, 16), 0)
    ones16 = jnp.where(row16 < _N, 1.0, 0.0).astype(jnp.float32)

    degp = _sc_aggregate(ones16, src3, dst3, zeros16, 16)
    xs1, dis = _tc_prep(degp, x_pad)
    acc1 = _sc_aggregate(xs1, src3, dst3, zeros128, _DIN)
    xs2 = _tc_mid(acc1, xs1, dis, W1, b1.reshape(1, _HID), W2)
    acc2 = _sc_aggregate(xs2, src3, dst3, zeros64, _OUT2)
    pooled, x4 = _tc_final(
        acc2, xs2, dis, b2.reshape(1, _OUT2), batch_row,
        fW1, fb1.reshape(1, _HID), fW2, fb2.reshape(1, _HID))
    return pooled, x4


# speedup vs baseline: 1.0637x; 1.0637x over previous
import functools

import jax
import jax.numpy as jnp
from jax import lax
from jax.experimental import pallas as pl
from jax.experimental.pallas import tpu as pltpu
from jax.experimental.pallas import tpu_sc as plsc

_N = 10000
_E = 320000
_DIN = 128
_HID = 256
_OUT2 = 64
_G = 64

_NP = 10240
_NW = 32
_K = 128
_EP = 327680
_ROWS = _EP // _K
_NBT = _ROWS // 16
_RPS = _NP // 16

_HIGH = lax.Precision.HIGHEST


def _sc_mesh():
    return plsc.VectorSubcoreMesh(core_axis_name="c", subcore_axis_name="s",
                                  num_cores=2, num_subcores=16)


_SC_PARAMS = pltpu.CompilerParams(use_tc_tiling_on_sc=False)


def _sc_aggregate(table, src2, dst2, zeros, d, nb0, hb):
    nb1 = _NBT - nb0
    assert nb0 % hb == 0 and nb1 % hb == 0

    @functools.partial(
        pl.kernel,
        out_type=jax.ShapeDtypeStruct((2, _NP, d), jnp.float32),
        mesh=_sc_mesh(),
        scratch_types=[
            pltpu.VMEM((hb, _K), jnp.int32),
            pltpu.VMEM((hb, _K), jnp.int32),
            pltpu.VMEM((_K, d), jnp.float32),
            pltpu.VMEM((_K, d), jnp.float32),
            pltpu.VMEM_SHARED((_NP, d), jnp.float32),
            pltpu.SemaphoreType.DMA,
            pltpu.SemaphoreType.DMA,
            pltpu.SemaphoreType.DMA,
            pltpu.SemaphoreType.DMA,
        ],
        compiler_params=_SC_PARAMS,
    )
    def agg_kernel(table_h, src_h, dst_h, zeros_h, out_h,
                   sidx, didx, rows0, rows1, acc, sg0, sg1, ss0, ss1):
        c = lax.axis_index("c")
        s = lax.axis_index("s")
        base = jnp.where(c == 0, s * nb0, 16 * nb0 + s * nb1)
        nh = jnp.where(c == 0, nb0 // hb, nb1 // hb)
        pltpu.sync_copy(zeros_h.at[pl.ds(s * _RPS, _RPS)],
                        acc.at[pl.ds(s * _RPS, _RPS)])
        plsc.subcore_barrier()

        def gs(b, buf, sem):
            pltpu.make_async_copy(table_h.at[sidx.at[b]], buf, sem).start()

        def gw(b, buf, sem):
            pltpu.make_async_copy(table_h.at[sidx.at[b]], buf, sem).wait()

        def ss(b, buf, sem):
            pltpu.async_copy(buf, acc.at[didx.at[b]], sem, add=True)

        def sw(b, buf, sem):
            pltpu.make_async_copy(buf, acc.at[didx.at[b]], sem).wait()

        @pl.loop(0, nh)
        def _(h):
            pltpu.sync_copy(src_h.at[pl.ds(base + h * hb, hb)], sidx)
            pltpu.sync_copy(dst_h.at[pl.ds(base + h * hb, hb)], didx)
            gs(0, rows0, sg0)

            @pl.loop(0, hb, step=2)
            def _(b):
                gs(b + 1, rows1, sg1)
                gw(b, rows0, sg0)
                ss(b, rows0, ss0)
                gw(b + 1, rows1, sg1)
                ss(b + 1, rows1, ss1)
                sw(b, rows0, ss0)

                @pl.when(b + 2 < hb)
                def _():
                    gs(b + 2, rows0, sg0)
                sw(b + 1, rows1, ss1)

        plsc.subcore_barrier()
        pltpu.sync_copy(acc.at[pl.ds(s * _RPS, _RPS)],
                        out_h.at[c, pl.ds(s * _RPS, _RPS)])

    return agg_kernel(table, src2, dst2, zeros)


def _tc_prep(degp, x_pad):
    def body(degp_ref, x_ref, xs1_ref, dis_ref):
        deg = degp_ref[0, :, 0:1] + degp_ref[1, :, 0:1] + 1.0
        dis = lax.rsqrt(deg)
        row = lax.broadcasted_iota(jnp.int32, (_NP, 1), 0)
        dis = jnp.where(row < _N, dis, 0.0)
        dis_ref[...] = dis
        xs1_ref[...] = x_ref[...] * dis

    return pl.pallas_call(
        body,
        out_shape=(jax.ShapeDtypeStruct((_NP, _DIN), jnp.float32),
                   jax.ShapeDtypeStruct((_NP, 1), jnp.float32)),
    )(degp, x_pad)


def _tc_mid(acc1, xs1, dis, W1, b1, W2):
    def body(acc_ref, xs1_ref, dis_ref, w1_ref, b1_ref, w2_ref, xs2_ref):
        dis = dis_ref[...]
        ax = dis * (acc_ref[0] + acc_ref[1] + xs1_ref[...])
        h1 = jnp.dot(ax, w1_ref[...], precision=_HIGH,
                     preferred_element_type=jnp.float32) + b1_ref[...]
        h1 = jnp.maximum(h1, 0.0)
        m2 = jnp.dot(h1, w2_ref[...], precision=_HIGH,
                     preferred_element_type=jnp.float32)
        xs2_ref[...] = dis * m2

    return pl.pallas_call(
        body,
        out_shape=jax.ShapeDtypeStruct((_NP, _OUT2), jnp.float32),
    )(acc1, xs1, dis, W1, b1, W2)


def _tc_final(acc2, xs2, dis, b2, batch_row, fW1, fb1, fW2, fb2):
    def body(acc_ref, xs2_ref, dis_ref, b2_ref, batch_ref,
             fw1_ref, fb1_ref, fw2_ref, fb2_ref, pooled_ref, x4_ref):
        dis = dis_ref[...]
        h2 = dis * (acc_ref[0] + acc_ref[1] + xs2_ref[...]) + b2_ref[...]
        h2 = jnp.maximum(h2, 0.0)
        gid = lax.broadcasted_iota(jnp.int32, (_G, _NP), 0)
        oneh = (gid == batch_ref[...]).astype(jnp.float32)
        psum = jnp.dot(oneh, h2, precision=_HIGH,
                       preferred_element_type=jnp.float32)
        cnt = jnp.sum(oneh, axis=1, keepdims=True)
        pooled = psum / jnp.maximum(cnt, 1.0)
        pooled_ref[...] = pooled
        x1 = jnp.dot(pooled, fw1_ref[...], precision=_HIGH,
                     preferred_element_type=jnp.float32) + fb1_ref[...]
        x2 = jnp.maximum(x1, 0.0)
        x3 = jnp.dot(x2, fw2_ref[...], precision=_HIGH,
                     preferred_element_type=jnp.float32) + fb2_ref[...]
        x4_ref[...] = jnp.maximum(x3, 0.0)

    return pl.pallas_call(
        body,
        out_shape=(jax.ShapeDtypeStruct((_G, _OUT2), jnp.float32),
                   jax.ShapeDtypeStruct((_G, _HID), jnp.float32)),
    )(acc2, xs2, dis, b2, batch_row, fW1, fb1, fW2, fb2)


def kernel(x, edge_index, batch, W1, b1, W2, b2, fW1, fb1, fW2, fb2):
    src = edge_index[0].astype(jnp.int32)
    dst = edge_index[1].astype(jnp.int32)
    epad = _EP - _E
    padv = jnp.full((epad,), _N, jnp.int32)
    src2 = jnp.concatenate([src, padv]).reshape(_ROWS, _K)
    dst2 = jnp.concatenate([dst, padv]).reshape(_ROWS, _K)

    x_pad = jnp.pad(x, ((0, _NP - _N), (0, 0)))
    batch_row = jnp.concatenate(
        [batch.astype(jnp.int32), jnp.full((_NP - _N,), _G, jnp.int32)]
    ).reshape(1, _NP)

    zeros16 = jnp.zeros((_NP, 16), jnp.float32)
    zeros128 = jnp.zeros((_NP, _DIN), jnp.float32)
    zeros64 = jnp.zeros((_NP, _OUT2), jnp.float32)
    row16 = lax.broadcasted_iota(jnp.int32, (_NP, 16), 0)
    ones16 = jnp.where(row16 < _N, 1.0, 0.0).astype(jnp.float32)

    degp = _sc_aggregate(ones16, src2, dst2, zeros16, 16, nb0=104, hb=8)
    xs1, dis = _tc_prep(degp, x_pad)
    acc1 = _sc_aggregate(xs1, src2, dst2, zeros128, _DIN, nb0=120, hb=40)
    xs2 = _tc_mid(acc1, xs1, dis, W1, b1.reshape(1, _HID), W2)
    acc2 = _sc_aggregate(xs2, src2, dst2, zeros64, _OUT2, nb0=112, hb=16)
    pooled, x4 = _tc_final(
        acc2, xs2, dis, b2.reshape(1, _OUT2), batch_row,
        fW1, fb1.reshape(1, _HID), fW2, fb2.reshape(1, _HID))
    return pooled, x4
